# Initial kernel scaffold; baseline (speedup 1.0000x reference)
import functools

import jax
import jax.numpy as jnp
from jax import lax
from jax.experimental import pallas as pl
from jax.experimental.pallas import tpu as pltpu
from jax.experimental.pallas import tpu_sc as plsc

_NUM_EMB = 8192
_DIM = 32
_N_TOK = 16 * 1024
_TOK = 256
_N_BLOCKS = _N_TOK // _TOK


def _argmin_body(z_ref, w_ref, idx_ref):
    z = z_ref[...]
    w = w_ref[...]
    s = lax.dot_general(
        z, w, (((1,), (1,)), ((), ())), preferred_element_type=jnp.float32
    )
    zsq = jnp.sum(z * z, axis=1, keepdims=True)
    wsq = jnp.sum(w * w, axis=1)
    dist = (zsq + wsq[None, :]) - 2.0 * s
    m = jnp.min(dist, axis=1, keepdims=True)
    ids = lax.broadcasted_iota(jnp.int32, dist.shape, 1)
    idx = jnp.min(jnp.where(dist == m, ids, _NUM_EMB), axis=1)
    idx_ref[0, 0, :] = idx


_argmin_call = pl.pallas_call(
    _argmin_body,
    grid=(_N_BLOCKS,),
    in_specs=[
        pl.BlockSpec((_TOK, _DIM), lambda i: (i, 0)),
        pl.BlockSpec((_NUM_EMB, _DIM), lambda i: (0, 0)),
    ],
    out_specs=pl.BlockSpec((1, 1, _TOK), lambda i: (i, 0, 0)),
    out_shape=jax.ShapeDtypeStruct((_N_BLOCKS, 1, _TOK), jnp.int32),
)


def _make_gather():
    info = plsc.get_sparse_core_info()
    nc, ns = info.num_cores, info.num_subcores
    nw = nc * ns
    b_per_w = _N_TOK // nw
    ch = 128

    @functools.partial(
        pl.kernel,
        mesh=plsc.VectorSubcoreMesh(core_axis_name="c", subcore_axis_name="s"),
        out_type=jax.ShapeDtypeStruct((_N_TOK, _DIM), jnp.float32),
        scratch_types=[
            pltpu.VMEM((b_per_w,), jnp.int32),
            pltpu.VMEM((b_per_w, _DIM), jnp.float32),
            pltpu.SemaphoreType.DMA,
        ],
    )
    def _gather(table_hbm, idx_hbm, out_hbm, idx_v, rows_v, sem):
        wid = lax.axis_index("s") * nc + lax.axis_index("c")
        base = wid * b_per_w
        pltpu.sync_copy(idx_hbm.at[pl.ds(base, b_per_w)], idx_v)
        copies = []
        for j in range(b_per_w // ch):
            copies.append(
                pltpu.async_copy(
                    table_hbm.at[idx_v.at[pl.ds(j * ch, ch)]],
                    rows_v.at[pl.ds(j * ch, ch)],
                    sem,
                )
            )
        for c in copies:
            c.wait()
        pltpu.sync_copy(rows_v, out_hbm.at[pl.ds(base, b_per_w)])

    return _gather


_gather_call = _make_gather()


def kernel(z_e, W):
    z_flat = z_e.reshape(_N_TOK, _DIM)
    idx = _argmin_call(z_flat, W).reshape(_N_TOK)
    rows = _gather_call(W, idx)
    return rows.reshape(z_e.shape)


# speedup vs baseline: 9.9004x; 9.9004x over previous
import functools

import jax
import jax.numpy as jnp
from jax import lax
from jax.experimental import pallas as pl
from jax.experimental.pallas import tpu as pltpu
from jax.experimental.pallas import tpu_sc as plsc

_NUM_EMB = 8192
_DIM = 32
_N_TOK = 16 * 1024
_TOK = 256
_N_BLOCKS = _N_TOK // _TOK


def _argmin_body(z_ref, w_ref, idx_ref):
    z = z_ref[...]
    w = w_ref[...]
    s = lax.dot_general(
        z.astype(jnp.bfloat16), w, (((1,), (1,)), ((), ())),
        preferred_element_type=jnp.float32,
        precision=lax.Precision.DEFAULT,
    )
    zsq = jnp.sum(z * z, axis=1, keepdims=True)
    wsq = jnp.sum(w * w, axis=1)
    dist = (zsq + wsq[None, :]) - 2.0 * s
    half = _NUM_EMB // 2
    d_a = dist[:, :half]
    d_b = dist[:, half:]
    ids = lax.broadcasted_iota(jnp.int32, d_a.shape, 1)
    m_a = jnp.min(d_a, axis=1, keepdims=True)
    i_a = jnp.min(jnp.where(d_a == m_a, ids, _NUM_EMB), axis=1)
    m_b = jnp.min(d_b, axis=1, keepdims=True)
    i_b = jnp.min(jnp.where(d_b == m_b, ids, _NUM_EMB), axis=1) + half
    m_a_bf = m_a[:, 0].astype(jnp.bfloat16).astype(jnp.float32)
    idx = jnp.where(m_a_bf <= m_b[:, 0], i_a, i_b)
    idx_ref[0, 0, :] = idx


_argmin_call = pl.pallas_call(
    _argmin_body,
    grid=(_N_BLOCKS,),
    in_specs=[
        pl.BlockSpec((_TOK, _DIM), lambda i: (i, 0)),
        pl.BlockSpec((_NUM_EMB, _DIM), lambda i: (0, 0)),
    ],
    out_specs=pl.BlockSpec((1, 1, _TOK), lambda i: (i, 0, 0)),
    out_shape=jax.ShapeDtypeStruct((_N_BLOCKS, 1, _TOK), jnp.int32),
)


def _make_gather():
    info = plsc.get_sparse_core_info()
    nc, ns = info.num_cores, info.num_subcores
    nw = nc * ns
    b_per_w = _N_TOK // nw
    ch = 128

    @functools.partial(
        pl.kernel,
        mesh=plsc.VectorSubcoreMesh(core_axis_name="c", subcore_axis_name="s"),
        compiler_params=pltpu.CompilerParams(use_tc_tiling_on_sc=False),
        out_type=jax.ShapeDtypeStruct((_N_TOK, _DIM), jnp.float32),
        scratch_types=[
            pltpu.VMEM((b_per_w,), jnp.int32),
            pltpu.VMEM((b_per_w, _DIM), jnp.float32),
            pltpu.SemaphoreType.DMA,
        ],
    )
    def _gather(table_hbm, idx_hbm, out_hbm, idx_v, rows_v, sem):
        wid = lax.axis_index("s") * nc + lax.axis_index("c")
        base = wid * b_per_w
        pltpu.sync_copy(idx_hbm.at[pl.ds(base, b_per_w)], idx_v)
        copies = []
        for j in range(b_per_w // ch):
            copies.append(
                pltpu.async_copy(
                    table_hbm.at[idx_v.at[pl.ds(j * ch, ch)]],
                    rows_v.at[pl.ds(j * ch, ch)],
                    sem,
                )
            )
        for c in copies:
            c.wait()
        pltpu.sync_copy(rows_v, out_hbm.at[pl.ds(base, b_per_w)])

    return _gather


_gather_call = _make_gather()


def kernel(z_e, W):
    z_flat = z_e.reshape(_N_TOK, _DIM)
    idx = _argmin_call(z_flat, W).reshape(_N_TOK)
    rows = _gather_call(W, idx)
    return rows.reshape(z_e.shape)
